# baseline (speedup 1.0000x reference)
import functools

import jax
import jax.numpy as jnp
from jax import lax
from jax.experimental import pallas as pl
from jax.experimental.pallas import tpu as pltpu
from jax.experimental.pallas import tpu_sc as plsc

N_NODES = 10000
N_PAD = 10240
IN_DIM = 128
HIDDEN_DIM = 64
OUT_DIM = 2
OUT_PAD = 8
N_EDGES = 320000

NC = 2
NS = 16
NW = NC * NS
CHUNK = 128
NBUF = 6
N_CHUNKS = 84
E_PAD = NW * N_CHUNKS * CHUNK
ROWS_PER_TILE = N_PAD // NS

_mesh = plsc.VectorSubcoreMesh(core_axis_name="c", subcore_axis_name="s")
_sc_params = pltpu.CompilerParams(use_tc_tiling_on_sc=False)


@functools.partial(
    pl.kernel,
    mesh=_mesh,
    out_type=jax.ShapeDtypeStruct((NC, N_PAD), jnp.float32),
    compiler_params=_sc_params,
    scratch_types=[
        pltpu.VMEM((N_CHUNKS, CHUNK), jnp.int32),
        pltpu.VMEM((CHUNK,), jnp.float32),
        pltpu.VMEM_SHARED((N_PAD,), jnp.float32),
        pltpu.SemaphoreType.DMA,
    ],
)
def _deg_kernel(dst_hbm, zeros_hbm, out_hbm, dst_v, ones_v, acc_sh, sem):
    cid = lax.axis_index("c")
    sid = lax.axis_index("s")
    wid = cid * NS + sid
    row0 = sid * ROWS_PER_TILE

    pltpu.sync_copy(zeros_hbm.at[pl.ds(row0, ROWS_PER_TILE)],
                    acc_sh.at[pl.ds(row0, ROWS_PER_TILE)])
    pltpu.sync_copy(dst_hbm.at[wid], dst_v)
    for i in range(CHUNK // 16):
        ones_v[pl.ds(i * 16, 16)] = jnp.ones((16,), jnp.float32)
    plsc.subcore_barrier()

    def fire(j, carry):
        pltpu.async_copy(ones_v, acc_sh.at[dst_v.at[j]], sem, add=True)
        return carry

    lax.fori_loop(0, N_CHUNKS, fire, 0)

    def drain(j, carry):
        pltpu.make_async_copy(ones_v, acc_sh.at[dst_v.at[0]], sem).wait()
        return carry

    lax.fori_loop(0, N_CHUNKS, drain, 0)
    plsc.subcore_barrier()
    pltpu.sync_copy(acc_sh.at[pl.ds(row0, ROWS_PER_TILE)],
                    out_hbm.at[cid, pl.ds(row0, ROWS_PER_TILE)])


def _make_agg_kernel(width):
    @functools.partial(
        pl.kernel,
        mesh=_mesh,
        out_type=jax.ShapeDtypeStruct((NC, N_PAD, width), jnp.float32),
        compiler_params=_sc_params,
        scratch_types=[
            pltpu.VMEM((N_CHUNKS, CHUNK), jnp.int32),
            pltpu.VMEM((N_CHUNKS, CHUNK), jnp.int32),
            pltpu.VMEM((NBUF, CHUNK, width), jnp.float32),
            pltpu.VMEM_SHARED((N_PAD, width), jnp.float32),
        ] + [pltpu.SemaphoreType.DMA] * (2 * NBUF),
    )
    def agg(y_hbm, src_hbm, dst_hbm, zeros_hbm, out_hbm,
            src_v, dst_v, rows_v, acc_sh, *sems):
        cid = lax.axis_index("c")
        sid = lax.axis_index("s")
        wid = cid * NS + sid
        row0 = sid * ROWS_PER_TILE
        sg = sems[:NBUF]
        ss = sems[NBUF:]

        pltpu.sync_copy(zeros_hbm.at[pl.ds(row0, ROWS_PER_TILE)],
                        acc_sh.at[pl.ds(row0, ROWS_PER_TILE)])
        pltpu.sync_copy(src_hbm.at[wid], src_v)
        pltpu.sync_copy(dst_hbm.at[wid], dst_v)
        plsc.subcore_barrier()

        for b in range(NBUF):
            pltpu.async_copy(y_hbm.at[src_v.at[b]], rows_v.at[b], sg[b])

        def body(i, carry):
            j = NBUF * i
            for b in range(NBUF):
                pltpu.make_async_copy(y_hbm.at[src_v.at[j + b]],
                                      rows_v.at[b], sg[b]).wait()
                pltpu.async_copy(rows_v.at[b], acc_sh.at[dst_v.at[j + b]],
                                 ss[b], add=True)
            for b in range(NBUF):
                pltpu.make_async_copy(rows_v.at[b], acc_sh.at[dst_v.at[j + b]],
                                      ss[b]).wait()
                pltpu.async_copy(y_hbm.at[src_v.at[j + NBUF + b]],
                                 rows_v.at[b], sg[b])
            return carry

        lax.fori_loop(0, (N_CHUNKS - NBUF) // NBUF, body, 0)

        for b in range(NBUF):
            j = N_CHUNKS - NBUF + b
            pltpu.make_async_copy(y_hbm.at[src_v.at[j]],
                                  rows_v.at[b], sg[b]).wait()
            pltpu.async_copy(rows_v.at[b], acc_sh.at[dst_v.at[j]],
                             ss[b], add=True)
        for b in range(NBUF):
            j = N_CHUNKS - NBUF + b
            pltpu.make_async_copy(rows_v.at[b], acc_sh.at[dst_v.at[j]],
                                  ss[b]).wait()

        plsc.subcore_barrier()
        pltpu.sync_copy(acc_sh.at[pl.ds(row0, ROWS_PER_TILE)],
                        out_hbm.at[cid, pl.ds(row0, ROWS_PER_TILE)])

    return agg


_agg64 = _make_agg_kernel(HIDDEN_DIM)
_agg8 = _make_agg_kernel(OUT_PAD)


_BR = 1000


def _k2_body(x_ref, w1_ref, degp_ref, y1_ref):
    h = jnp.dot(x_ref[...], w1_ref[...], preferred_element_type=jnp.float32)
    d = lax.rsqrt(degp_ref[0] + degp_ref[1])
    y1_ref[...] = h * d


def _k4_body(aggp_ref, degp_ref, w2_ref, b1_ref, y2_ref):
    d = lax.rsqrt(degp_ref[0] + degp_ref[1])
    s = aggp_ref[0] + aggp_ref[1]
    out1 = jnp.maximum(d * s + b1_ref[...], 0.0)
    h2 = jnp.dot(out1, w2_ref[...], preferred_element_type=jnp.float32)
    y2_ref[...] = h2 * d


def _k6_body(aggp_ref, degp_ref, b2_ref, out_ref):
    d = lax.rsqrt(degp_ref[0] + degp_ref[1])
    s = aggp_ref[0] + aggp_ref[1]
    r = d * s + b2_ref[...]
    out_ref[...] = r[:, :OUT_DIM]


def kernel(x, edge_index, W1, b1, W2, b2):
    src = edge_index[0].astype(jnp.int32)
    dst = edge_index[1].astype(jnp.int32)
    n_dummy = E_PAD - N_EDGES - N_NODES
    loop = jnp.arange(N_NODES, dtype=jnp.int32)
    dummy_dst = N_NODES + (jnp.arange(n_dummy, dtype=jnp.int32)
                           % (N_PAD - N_NODES))
    src3 = jnp.concatenate(
        [src, loop, jnp.zeros((n_dummy,), jnp.int32)]).reshape(
            N_CHUNKS * CHUNK, NW).T.reshape(NW, N_CHUNKS, CHUNK)
    dst3 = jnp.concatenate(
        [dst, loop, dummy_dst]).reshape(
            N_CHUNKS * CHUNK, NW).T.reshape(NW, N_CHUNKS, CHUNK)

    z1 = jnp.zeros((N_PAD,), jnp.float32)
    z64 = jnp.zeros((N_PAD, HIDDEN_DIM), jnp.float32)
    z8 = jnp.zeros((N_PAD, OUT_PAD), jnp.float32)
    w2p = jnp.pad(W2, ((0, 0), (0, OUT_PAD - OUT_DIM)))
    b2p = jnp.pad(b2, (0, OUT_PAD - OUT_DIM)).reshape(1, OUT_PAD)

    degp = _deg_kernel(dst3, z1)
    degp3 = degp.reshape(NC, N_PAD, 1)

    y1 = pl.pallas_call(
        _k2_body,
        grid=(N_NODES // _BR,),
        in_specs=[
            pl.BlockSpec((_BR, IN_DIM), lambda i: (i, 0)),
            pl.BlockSpec((IN_DIM, HIDDEN_DIM), lambda i: (0, 0)),
            pl.BlockSpec((NC, _BR, 1), lambda i: (0, i, 0)),
        ],
        out_specs=pl.BlockSpec((_BR, HIDDEN_DIM), lambda i: (i, 0)),
        out_shape=jax.ShapeDtypeStruct((N_NODES, HIDDEN_DIM), jnp.float32),
    )(x, W1, degp3)

    aggp1 = _agg64(y1, src3, dst3, z64)

    y2 = pl.pallas_call(
        _k4_body,
        grid=(N_NODES // _BR,),
        in_specs=[
            pl.BlockSpec((NC, _BR, HIDDEN_DIM), lambda i: (0, i, 0)),
            pl.BlockSpec((NC, _BR, 1), lambda i: (0, i, 0)),
            pl.BlockSpec((HIDDEN_DIM, OUT_PAD), lambda i: (0, 0)),
            pl.BlockSpec((1, HIDDEN_DIM), lambda i: (0, 0)),
        ],
        out_specs=pl.BlockSpec((_BR, OUT_PAD), lambda i: (i, 0)),
        out_shape=jax.ShapeDtypeStruct((N_NODES, OUT_PAD), jnp.float32),
    )(aggp1, degp3, w2p, b1.reshape(1, HIDDEN_DIM))

    aggp2 = _agg8(y2, src3, dst3, z8)

    out = pl.pallas_call(
        _k6_body,
        grid=(N_NODES // _BR,),
        in_specs=[
            pl.BlockSpec((NC, _BR, OUT_PAD), lambda i: (0, i, 0)),
            pl.BlockSpec((NC, _BR, 1), lambda i: (0, i, 0)),
            pl.BlockSpec((1, OUT_PAD), lambda i: (0, 0)),
        ],
        out_specs=pl.BlockSpec((_BR, OUT_DIM), lambda i: (i, 0)),
        out_shape=jax.ShapeDtypeStruct((N_NODES, OUT_DIM), jnp.float32),
    )(aggp2, degp3, b2p)

    return out


# speedup vs baseline: 1.0498x; 1.0498x over previous
import functools

import jax
import jax.numpy as jnp
from jax import lax
from jax.experimental import pallas as pl
from jax.experimental.pallas import tpu as pltpu
from jax.experimental.pallas import tpu_sc as plsc

N_NODES = 10000
N_PAD = 10240
IN_DIM = 128
HIDDEN_DIM = 64
OUT_DIM = 2
OUT_PAD = 8
N_EDGES = 320000

NC = 2
NS = 16
NW = NC * NS
CHUNK = 128
NBUF = 6
N_CHUNKS = 84
E_PAD = NW * N_CHUNKS * CHUNK
ROWS_PER_TILE = N_PAD // NS

_mesh = plsc.VectorSubcoreMesh(core_axis_name="c", subcore_axis_name="s")
_sc_params = pltpu.CompilerParams(use_tc_tiling_on_sc=False)


@functools.partial(
    pl.kernel,
    mesh=_mesh,
    out_type=jax.ShapeDtypeStruct((NC, N_PAD), jnp.float32),
    compiler_params=_sc_params,
    scratch_types=[
        pltpu.VMEM((N_CHUNKS, CHUNK), jnp.int32),
        pltpu.VMEM((CHUNK,), jnp.float32),
        pltpu.VMEM_SHARED((N_PAD,), jnp.float32),
        pltpu.SemaphoreType.DMA,
    ],
)
def _deg_kernel(dst_hbm, zeros_hbm, out_hbm, dst_v, ones_v, acc_sh, sem):
    cid = lax.axis_index("c")
    sid = lax.axis_index("s")
    wid = cid * NS + sid
    row0 = sid * ROWS_PER_TILE

    pltpu.sync_copy(zeros_hbm.at[pl.ds(row0, ROWS_PER_TILE)],
                    acc_sh.at[pl.ds(row0, ROWS_PER_TILE)])
    pltpu.sync_copy(dst_hbm.at[wid], dst_v)
    for i in range(CHUNK // 16):
        ones_v[pl.ds(i * 16, 16)] = jnp.ones((16,), jnp.float32)
    plsc.subcore_barrier()

    def fire(j, carry):
        pltpu.async_copy(ones_v, acc_sh.at[dst_v.at[j]], sem, add=True)
        return carry

    lax.fori_loop(0, N_CHUNKS, fire, 0)

    def drain(j, carry):
        pltpu.make_async_copy(ones_v, acc_sh.at[dst_v.at[0]], sem).wait()
        return carry

    lax.fori_loop(0, N_CHUNKS, drain, 0)
    plsc.subcore_barrier()
    pltpu.sync_copy(acc_sh.at[pl.ds(row0, ROWS_PER_TILE)],
                    out_hbm.at[cid, pl.ds(row0, ROWS_PER_TILE)])


def _make_agg_kernel(width):
    @functools.partial(
        pl.kernel,
        mesh=_mesh,
        out_type=jax.ShapeDtypeStruct((NC, N_PAD, width), jnp.float32),
        compiler_params=_sc_params,
        scratch_types=[
            pltpu.VMEM((N_CHUNKS, CHUNK), jnp.int32),
            pltpu.VMEM((N_CHUNKS, CHUNK), jnp.int32),
            pltpu.VMEM((NBUF, CHUNK, width), jnp.float32),
            pltpu.VMEM_SHARED((N_PAD, width), jnp.float32),
        ] + [pltpu.SemaphoreType.DMA] * (2 * NBUF),
    )
    def agg(y_hbm, src_hbm, dst_hbm, zeros_hbm, out_hbm,
            src_v, dst_v, rows_v, acc_sh, *sems):
        cid = lax.axis_index("c")
        sid = lax.axis_index("s")
        wid = cid * NS + sid
        row0 = sid * ROWS_PER_TILE
        sg = sems[:NBUF]
        ss = sems[NBUF:]

        pltpu.sync_copy(zeros_hbm.at[pl.ds(row0, ROWS_PER_TILE)],
                        acc_sh.at[pl.ds(row0, ROWS_PER_TILE)])
        pltpu.sync_copy(src_hbm.at[wid], src_v)
        pltpu.sync_copy(dst_hbm.at[wid], dst_v)
        plsc.subcore_barrier()

        for b in range(NBUF):
            pltpu.async_copy(y_hbm.at[src_v.at[b]], rows_v.at[b], sg[b])

        def body(i, carry):
            j = NBUF * i
            for b in range(NBUF):
                pltpu.make_async_copy(y_hbm.at[src_v.at[j + b]],
                                      rows_v.at[b], sg[b]).wait()
                pltpu.async_copy(rows_v.at[b], acc_sh.at[dst_v.at[j + b]],
                                 ss[b], add=True)
            for b in range(NBUF):
                pltpu.make_async_copy(rows_v.at[b], acc_sh.at[dst_v.at[j + b]],
                                      ss[b]).wait()
                pltpu.async_copy(y_hbm.at[src_v.at[j + NBUF + b]],
                                 rows_v.at[b], sg[b])
            return carry

        lax.fori_loop(0, (N_CHUNKS - NBUF) // NBUF, body, 0)

        for b in range(NBUF):
            j = N_CHUNKS - NBUF + b
            pltpu.make_async_copy(y_hbm.at[src_v.at[j]],
                                  rows_v.at[b], sg[b]).wait()
            pltpu.async_copy(rows_v.at[b], acc_sh.at[dst_v.at[j]],
                             ss[b], add=True)
        for b in range(NBUF):
            j = N_CHUNKS - NBUF + b
            pltpu.make_async_copy(rows_v.at[b], acc_sh.at[dst_v.at[j]],
                                  ss[b]).wait()

        plsc.subcore_barrier()
        pltpu.sync_copy(acc_sh.at[pl.ds(row0, ROWS_PER_TILE)],
                        out_hbm.at[cid, pl.ds(row0, ROWS_PER_TILE)])

    return agg


_agg64 = _make_agg_kernel(HIDDEN_DIM)
_agg8 = _make_agg_kernel(OUT_PAD)


_BR = 1000
_BR2 = 1024


def _k2_body(x_ref, w1_ref, degp_ref, y1_ref):
    h = jnp.dot(x_ref[...], w1_ref[...], preferred_element_type=jnp.float32)
    d = lax.rsqrt(degp_ref[0] + degp_ref[1])
    y1_ref[...] = h * d


def _k4_body(aggp_ref, degp_ref, w2_ref, b1_ref, y2_ref):
    d = lax.rsqrt(degp_ref[0] + degp_ref[1])
    s = aggp_ref[0] + aggp_ref[1]
    out1 = jnp.maximum(d * s + b1_ref[...], 0.0)
    h2 = jnp.dot(out1, w2_ref[...], preferred_element_type=jnp.float32)
    y2_ref[...] = h2 * d


def _k6_body(aggp_ref, degp_ref, b2_ref, out_ref):
    d = lax.rsqrt(degp_ref[0] + degp_ref[1])
    s = aggp_ref[0] + aggp_ref[1]
    r = d * s + b2_ref[...]
    out_ref[...] = r[:, :OUT_DIM]


def kernel(x, edge_index, W1, b1, W2, b2):
    src = edge_index[0].astype(jnp.int32)
    dst = edge_index[1].astype(jnp.int32)
    n_dummy_w = N_CHUNKS * CHUNK - N_EDGES // NW - N_PAD // NW
    loop = jnp.arange(N_PAD, dtype=jnp.int32).reshape(NW, N_PAD // NW)
    dummy_src = jnp.zeros((NW, n_dummy_w), jnp.int32)
    dummy_dst = jnp.broadcast_to(
        N_NODES + (jnp.arange(n_dummy_w, dtype=jnp.int32)
                   % (N_PAD - N_NODES))[None, :], (NW, n_dummy_w))
    src3 = jnp.concatenate(
        [src.reshape(NW, N_EDGES // NW), loop, dummy_src],
        axis=1).reshape(NW, N_CHUNKS, CHUNK)
    dst3 = jnp.concatenate(
        [dst.reshape(NW, N_EDGES // NW), loop, dummy_dst],
        axis=1).reshape(NW, N_CHUNKS, CHUNK)
    xp = jnp.pad(x, ((0, N_PAD - N_NODES), (0, 0)))

    z1 = jnp.zeros((N_PAD,), jnp.float32)
    z64 = jnp.zeros((N_PAD, HIDDEN_DIM), jnp.float32)
    z8 = jnp.zeros((N_PAD, OUT_PAD), jnp.float32)
    w2p = jnp.pad(W2, ((0, 0), (0, OUT_PAD - OUT_DIM)))
    b2p = jnp.pad(b2, (0, OUT_PAD - OUT_DIM)).reshape(1, OUT_PAD)

    degp = _deg_kernel(dst3, z1)
    degp3 = degp.reshape(NC, N_PAD, 1)

    y1 = pl.pallas_call(
        _k2_body,
        grid=(N_PAD // _BR2,),
        in_specs=[
            pl.BlockSpec((_BR2, IN_DIM), lambda i: (i, 0)),
            pl.BlockSpec((IN_DIM, HIDDEN_DIM), lambda i: (0, 0)),
            pl.BlockSpec((NC, _BR2, 1), lambda i: (0, i, 0)),
        ],
        out_specs=pl.BlockSpec((_BR2, HIDDEN_DIM), lambda i: (i, 0)),
        out_shape=jax.ShapeDtypeStruct((N_PAD, HIDDEN_DIM), jnp.float32),
    )(xp, W1, degp3)

    aggp1 = _agg64(y1, src3, dst3, z64)

    y2 = pl.pallas_call(
        _k4_body,
        grid=(N_PAD // _BR2,),
        in_specs=[
            pl.BlockSpec((NC, _BR2, HIDDEN_DIM), lambda i: (0, i, 0)),
            pl.BlockSpec((NC, _BR2, 1), lambda i: (0, i, 0)),
            pl.BlockSpec((HIDDEN_DIM, OUT_PAD), lambda i: (0, 0)),
            pl.BlockSpec((1, HIDDEN_DIM), lambda i: (0, 0)),
        ],
        out_specs=pl.BlockSpec((_BR2, OUT_PAD), lambda i: (i, 0)),
        out_shape=jax.ShapeDtypeStruct((N_PAD, OUT_PAD), jnp.float32),
    )(aggp1, degp3, w2p, b1.reshape(1, HIDDEN_DIM))

    aggp2 = _agg8(y2, src3, dst3, z8)

    out = pl.pallas_call(
        _k6_body,
        grid=(N_NODES // _BR,),
        in_specs=[
            pl.BlockSpec((NC, _BR, OUT_PAD), lambda i: (0, i, 0)),
            pl.BlockSpec((NC, _BR, 1), lambda i: (0, i, 0)),
            pl.BlockSpec((1, OUT_PAD), lambda i: (0, 0)),
        ],
        out_specs=pl.BlockSpec((_BR, OUT_DIM), lambda i: (i, 0)),
        out_shape=jax.ShapeDtypeStruct((N_NODES, OUT_DIM), jnp.float32),
    )(aggp2, degp3, b2p)

    return out


# speedup vs baseline: 2.6605x; 2.5342x over previous
import functools

import jax
import jax.numpy as jnp
from jax import lax
from jax.experimental import pallas as pl
from jax.experimental.pallas import tpu as pltpu
from jax.experimental.pallas import tpu_sc as plsc

N_NODES = 10000
N_PAD = 10240
IN_DIM = 128
HIDDEN_DIM = 64
OUT_DIM = 2
OUT_PAD = 8
N_EDGES = 320000

NC = 2
NS = 16
NW = NC * NS
CHUNK = 128
NBUF = 6
N_CHUNKS = 84
E_PAD = NW * N_CHUNKS * CHUNK
ROWS_PER_TILE = N_PAD // NS

_mesh = plsc.VectorSubcoreMesh(core_axis_name="c", subcore_axis_name="s")
_sc_params = pltpu.CompilerParams(use_tc_tiling_on_sc=False)


@functools.partial(
    pl.kernel,
    mesh=_mesh,
    out_type=jax.ShapeDtypeStruct((NC, N_PAD), jnp.float32),
    compiler_params=_sc_params,
    scratch_types=[
        pltpu.VMEM((N_CHUNKS, CHUNK), jnp.int32),
        pltpu.VMEM((CHUNK,), jnp.float32),
        pltpu.VMEM_SHARED((N_PAD,), jnp.float32),
        pltpu.SemaphoreType.DMA,
    ],
)
def _deg_kernel(dst_hbm, zeros_hbm, out_hbm, dst_v, ones_v, acc_sh, sem):
    cid = lax.axis_index("c")
    sid = lax.axis_index("s")
    wid = cid * NS + sid
    row0 = sid * ROWS_PER_TILE

    pltpu.sync_copy(zeros_hbm.at[pl.ds(row0, ROWS_PER_TILE)],
                    acc_sh.at[pl.ds(row0, ROWS_PER_TILE)])
    pltpu.sync_copy(dst_hbm.at[wid], dst_v)
    for i in range(CHUNK // 16):
        ones_v[pl.ds(i * 16, 16)] = jnp.ones((16,), jnp.float32)
    plsc.subcore_barrier()

    def fire(j, carry):
        pltpu.async_copy(ones_v, acc_sh.at[dst_v.at[j]], sem, add=True)
        return carry

    lax.fori_loop(0, N_CHUNKS, fire, 0)

    def drain(j, carry):
        pltpu.make_async_copy(ones_v, acc_sh.at[dst_v.at[0]], sem).wait()
        return carry

    lax.fori_loop(0, N_CHUNKS, drain, 0)
    plsc.subcore_barrier()
    pltpu.sync_copy(acc_sh.at[pl.ds(row0, ROWS_PER_TILE)],
                    out_hbm.at[cid, pl.ds(row0, ROWS_PER_TILE)])


def _make_agg_kernel(width):
    @functools.partial(
        pl.kernel,
        mesh=_mesh,
        out_type=jax.ShapeDtypeStruct((NC, N_PAD, width), jnp.float32),
        compiler_params=_sc_params,
        scratch_types=[
            pltpu.VMEM((N_CHUNKS, CHUNK), jnp.int32),
            pltpu.VMEM((N_CHUNKS, CHUNK), jnp.int32),
            pltpu.VMEM((NBUF, CHUNK, width), jnp.float32),
            pltpu.VMEM_SHARED((N_PAD, width), jnp.float32),
        ] + [pltpu.SemaphoreType.DMA] * (2 * NBUF),
    )
    def agg(y_hbm, src_hbm, dst_hbm, zeros_hbm, out_hbm,
            src_v, dst_v, rows_v, acc_sh, *sems):
        cid = lax.axis_index("c")
        sid = lax.axis_index("s")
        wid = cid * NS + sid
        row0 = sid * ROWS_PER_TILE
        sg = sems[:NBUF]
        ss = sems[NBUF:]

        pltpu.sync_copy(zeros_hbm.at[pl.ds(row0, ROWS_PER_TILE)],
                        acc_sh.at[pl.ds(row0, ROWS_PER_TILE)])
        pltpu.sync_copy(src_hbm.at[wid], src_v)
        pltpu.sync_copy(dst_hbm.at[wid], dst_v)
        plsc.subcore_barrier()

        for b in range(NBUF):
            pltpu.async_copy(y_hbm.at[src_v.at[b]], rows_v.at[b], sg[b])

        def body(i, carry):
            j = NBUF * i
            for b in range(NBUF):
                pltpu.make_async_copy(y_hbm.at[src_v.at[j + b]],
                                      rows_v.at[b], sg[b]).wait()
                pltpu.async_copy(rows_v.at[b], acc_sh.at[dst_v.at[j + b]],
                                 ss[b], add=True)
            for b in range(NBUF):
                pltpu.make_async_copy(rows_v.at[b], acc_sh.at[dst_v.at[j + b]],
                                      ss[b]).wait()
                pltpu.async_copy(y_hbm.at[src_v.at[j + NBUF + b]],
                                 rows_v.at[b], sg[b])
            return carry

        lax.fori_loop(0, (N_CHUNKS - NBUF) // NBUF, body, 0)

        for b in range(NBUF):
            j = N_CHUNKS - NBUF + b
            pltpu.make_async_copy(y_hbm.at[src_v.at[j]],
                                  rows_v.at[b], sg[b]).wait()
            pltpu.async_copy(rows_v.at[b], acc_sh.at[dst_v.at[j]],
                             ss[b], add=True)
        for b in range(NBUF):
            j = N_CHUNKS - NBUF + b
            pltpu.make_async_copy(rows_v.at[b], acc_sh.at[dst_v.at[j]],
                                  ss[b]).wait()

        plsc.subcore_barrier()
        pltpu.sync_copy(acc_sh.at[pl.ds(row0, ROWS_PER_TILE)],
                        out_hbm.at[cid, pl.ds(row0, ROWS_PER_TILE)])

    return agg


_agg64 = _make_agg_kernel(HIDDEN_DIM)
_agg8 = _make_agg_kernel(OUT_PAD)


_BR = 1000
_BR2 = 1024


def _k2_body(x_ref, w1_ref, degp_ref, y1_ref):
    h = jnp.dot(x_ref[...], w1_ref[...], preferred_element_type=jnp.float32)
    d = lax.rsqrt(degp_ref[0] + degp_ref[1])
    y1_ref[...] = h * d


def _k4_body(aggp_ref, degp_ref, w2_ref, b1_ref, y2_ref):
    d = lax.rsqrt(degp_ref[0] + degp_ref[1])
    s = aggp_ref[0] + aggp_ref[1]
    out1 = jnp.maximum(d * s + b1_ref[...], 0.0)
    h2 = jnp.dot(out1, w2_ref[...], preferred_element_type=jnp.float32)
    y2_ref[...] = h2 * d


def _k6_body(aggp_ref, degp_ref, b2_ref, out_ref):
    d = lax.rsqrt(degp_ref[0] + degp_ref[1])
    s = aggp_ref[0] + aggp_ref[1]
    r = d * s + b2_ref[...]
    out_ref[...] = r[:, :OUT_DIM]


def kernel(x, edge_index, W1, b1, W2, b2):
    src = edge_index[0].astype(jnp.int32)
    dst = edge_index[1].astype(jnp.int32)
    n_dummy_w = N_CHUNKS * CHUNK - N_EDGES // NW - N_PAD // NW
    loop = jnp.arange(N_PAD, dtype=jnp.int32).reshape(NW, N_PAD // NW)
    dummy_src = jnp.broadcast_to(
        ((jnp.arange(n_dummy_w, dtype=jnp.int32) * 16) % N_NODES)[None, :],
        (NW, n_dummy_w))
    dummy_dst = jnp.broadcast_to(
        N_NODES + (jnp.arange(n_dummy_w, dtype=jnp.int32)
                   % (N_PAD - N_NODES))[None, :], (NW, n_dummy_w))
    src3 = jnp.concatenate(
        [src.reshape(NW, N_EDGES // NW), loop, dummy_src],
        axis=1).reshape(NW, N_CHUNKS, CHUNK)
    dst3 = jnp.concatenate(
        [dst.reshape(NW, N_EDGES // NW), loop, dummy_dst],
        axis=1).reshape(NW, N_CHUNKS, CHUNK)
    xp = jnp.pad(x, ((0, N_PAD - N_NODES), (0, 0)))

    z1 = jnp.zeros((N_PAD,), jnp.float32)
    z64 = jnp.zeros((N_PAD, HIDDEN_DIM), jnp.float32)
    z8 = jnp.zeros((N_PAD, OUT_PAD), jnp.float32)
    w2p = jnp.pad(W2, ((0, 0), (0, OUT_PAD - OUT_DIM)))
    b2p = jnp.pad(b2, (0, OUT_PAD - OUT_DIM)).reshape(1, OUT_PAD)

    degp = _deg_kernel(dst3, z1)
    degp3 = degp.reshape(NC, N_PAD, 1)

    y1 = pl.pallas_call(
        _k2_body,
        grid=(N_PAD // _BR2,),
        in_specs=[
            pl.BlockSpec((_BR2, IN_DIM), lambda i: (i, 0)),
            pl.BlockSpec((IN_DIM, HIDDEN_DIM), lambda i: (0, 0)),
            pl.BlockSpec((NC, _BR2, 1), lambda i: (0, i, 0)),
        ],
        out_specs=pl.BlockSpec((_BR2, HIDDEN_DIM), lambda i: (i, 0)),
        out_shape=jax.ShapeDtypeStruct((N_PAD, HIDDEN_DIM), jnp.float32),
    )(xp, W1, degp3)

    aggp1 = _agg64(y1, src3, dst3, z64)

    y2 = pl.pallas_call(
        _k4_body,
        grid=(N_PAD // _BR2,),
        in_specs=[
            pl.BlockSpec((NC, _BR2, HIDDEN_DIM), lambda i: (0, i, 0)),
            pl.BlockSpec((NC, _BR2, 1), lambda i: (0, i, 0)),
            pl.BlockSpec((HIDDEN_DIM, OUT_PAD), lambda i: (0, 0)),
            pl.BlockSpec((1, HIDDEN_DIM), lambda i: (0, 0)),
        ],
        out_specs=pl.BlockSpec((_BR2, OUT_PAD), lambda i: (i, 0)),
        out_shape=jax.ShapeDtypeStruct((N_PAD, OUT_PAD), jnp.float32),
    )(aggp1, degp3, w2p, b1.reshape(1, HIDDEN_DIM))

    aggp2 = _agg8(y2, src3, dst3, z8)

    out = pl.pallas_call(
        _k6_body,
        grid=(N_NODES // _BR,),
        in_specs=[
            pl.BlockSpec((NC, _BR, OUT_PAD), lambda i: (0, i, 0)),
            pl.BlockSpec((NC, _BR, 1), lambda i: (0, i, 0)),
            pl.BlockSpec((1, OUT_PAD), lambda i: (0, 0)),
        ],
        out_specs=pl.BlockSpec((_BR, OUT_DIM), lambda i: (i, 0)),
        out_shape=jax.ShapeDtypeStruct((N_NODES, OUT_DIM), jnp.float32),
    )(aggp2, degp3, b2p)

    return out
